# Initial kernel scaffold; baseline (speedup 1.0000x reference)
import functools

import jax
import jax.numpy as jnp
from jax import lax
from jax.experimental import pallas as pl
from jax.experimental.pallas import tpu as pltpu
from jax.experimental.pallas import tpu_sc as plsc

_BINS = (1, 2, 3, 4, 8, 16, 32, 64)

_NC = 2
_NS = 16
_L = 16


def kernel(lengths, table):
    n = lengths.shape[0]
    rows, d = table.shape
    nw = _NC * _NS
    n_per_w = n // nw
    groups = n_per_w // _L

    mesh = plsc.VectorSubcoreMesh(
        core_axis_name="c", subcore_axis_name="s",
        num_cores=_NC, num_subcores=_NS)

    @functools.partial(
        pl.kernel,
        out_type=jax.ShapeDtypeStruct((n, d), jnp.float32),
        mesh=mesh,
        scratch_types=[
            pltpu.VMEM((n_per_w,), jnp.int32),
            pltpu.VMEM((rows, d), jnp.float32),
            pltpu.VMEM((n_per_w, d), jnp.float32),
        ],
    )
    def run(lengths_hbm, table_hbm, out_hbm, len_v, tab_v, out_v):
        wid = lax.axis_index("s") * _NC + lax.axis_index("c")
        base = wid * n_per_w
        pltpu.sync_copy(lengths_hbm.at[pl.ds(base, n_per_w)], len_v)
        pltpu.sync_copy(table_hbm, tab_v)

        lane = lax.iota(jnp.int32, _L)

        def body(g, carry):
            lv = len_v[pl.ds(g * _L, _L)]
            idx = jnp.zeros((_L,), jnp.int32)
            for b in _BINS:
                idx = idx + (lv >= b).astype(jnp.int32)
            row_ids = g * _L + lane
            for col in range(d):
                col_ids = jnp.full((_L,), col, jnp.int32)
                vals = plsc.load_gather(tab_v, [idx, col_ids])
                plsc.store_scatter(out_v, [row_ids, col_ids], vals)
            return carry

        lax.fori_loop(0, groups, body, 0)
        pltpu.sync_copy(out_v, out_hbm.at[pl.ds(base, n_per_w)])

    return run(lengths, table)


# speedup vs baseline: 1.7527x; 1.7527x over previous
import functools

import jax
import jax.numpy as jnp
from jax import lax
from jax.experimental import pallas as pl
from jax.experimental.pallas import tpu as pltpu
from jax.experimental.pallas import tpu_sc as plsc

_BINS = (1, 2, 3, 4, 8, 16, 32, 64)
_NC, _NS, _L = 2, 16, 16


def kernel(lengths, table):
    n = lengths.shape[0]
    rows, d = table.shape
    nw = _NC * _NS
    n_per_w = n // nw
    groups = n_per_w // _L

    mesh = plsc.VectorSubcoreMesh(
        core_axis_name="c", subcore_axis_name="s",
        num_cores=_NC, num_subcores=_NS)

    @functools.partial(
        pl.kernel,
        out_type=jax.ShapeDtypeStruct((n * d,), jnp.float32),
        mesh=mesh,
        compiler_params=pltpu.CompilerParams(needs_layout_passes=False),
        scratch_types=[
            pltpu.VMEM((n_per_w,), jnp.int32),
            pltpu.VMEM((rows * d,), jnp.float32),
            pltpu.VMEM((n_per_w * d,), jnp.float32),
        ],
    )
    def run(lengths_hbm, table_hbm, out_hbm, len_v, tab_v, out_v):
        wid = lax.axis_index("s") * _NC + lax.axis_index("c")
        base = wid * n_per_w
        pltpu.sync_copy(lengths_hbm.at[pl.ds(base, n_per_w)], len_v)
        pltpu.sync_copy(table_hbm, tab_v)

        lane = lax.iota(jnp.int32, _L)

        def body(g, carry):
            lv = len_v[pl.ds(g * _L, _L)]
            idx = jnp.zeros((_L,), jnp.int32)
            for b in _BINS:
                idx = idx + (lv >= b).astype(jnp.int32)
            tpos = idx * d
            opos = (g * _L + lane) * d
            for col in range(d):
                vals = plsc.load_gather(tab_v, [tpos + col])
                plsc.store_scatter(out_v, [opos + col], vals)
            return carry

        lax.fori_loop(0, groups, body, 0)
        pltpu.sync_copy(out_v, out_hbm.at[pl.ds(base * d, n_per_w * d)])

    return run(lengths, table.reshape(-1)).reshape(n, d)
